# baseline (speedup 1.0000x reference)
import functools

import jax
import jax.numpy as jnp
from jax import lax
from jax.experimental import pallas as pl
from jax.experimental.pallas import tpu as pltpu
from jax.experimental.pallas import tpu_sc as plsc

_CHUNK = 128
_TW = 2048
_TH = 25 * _TW


def _tc_pack_pair(ta, tb):
  G = _TH // _TW
  last_blk = (ta.shape[1] - 1) // _TW

  def body(a_lo, a_hi, b_lo, b_hi, out_r):
    out_r[0, :, :] = jnp.concatenate([a_lo[...].T, a_hi[...].T], axis=1)
    out_r[1, :, :] = jnp.concatenate([b_lo[...].T, b_hi[...].T], axis=1)

  lo_spec = pl.BlockSpec((64, _TW), lambda i: (0, i))
  hi_spec = pl.BlockSpec((64, _TW), lambda i: (0, jnp.minimum(i + G, last_blk)))
  return pl.pallas_call(
      body,
      grid=(G,),
      in_specs=[lo_spec, hi_spec, lo_spec, hi_spec],
      out_specs=pl.BlockSpec((2, _TW, 128), lambda i: (0, i, 0)),
      out_shape=jax.ShapeDtypeStruct((2, _TH, 128), jnp.float32),
  )(ta, ta, tb, tb)


def _sc_gather_pair(cidx, table):
  B2 = cidx.shape[0]
  D = table.shape[1]
  info = plsc.get_sparse_core_info()
  NC, NS = info.num_cores, info.num_subcores
  NW = NC * NS
  assert B2 % (NW * _CHUNK) == 0
  bpw = B2 // NW
  ch = bpw // _CHUNK

  cidx2 = cidx.reshape(NW * ch, _CHUNK)

  mesh = plsc.VectorSubcoreMesh(core_axis_name="c", subcore_axis_name="s")

  @functools.partial(
      pl.kernel,
      out_type=jax.ShapeDtypeStruct((B2, D), jnp.float32),
      mesh=mesh,
      compiler_params=pltpu.CompilerParams(use_tc_tiling_on_sc=False),
      scratch_types=[
          pltpu.VMEM((ch, _CHUNK), jnp.int32),
          pltpu.VMEM((bpw, D), jnp.float32),
          pltpu.SemaphoreType.DMA,
      ],
  )
  def k(cix, tbl, out, idx_v, rows, sem):
    wid = lax.axis_index("s") * NC + lax.axis_index("c")
    pltpu.sync_copy(cix.at[pl.ds(wid * ch, ch)], idx_v)
    descs = [
        pltpu.async_copy(tbl.at[idx_v.at[j]],
                         rows.at[pl.ds(j * _CHUNK, _CHUNK)], sem)
        for j in range(ch)
    ]
    for d in descs:
      d.wait()
    pltpu.sync_copy(rows, out.at[pl.ds(wid * bpw, bpw)])

  return k(cidx2, table)


def _tc_mlp(GG, XX, W1, b1, W2, b2, W3, b3, Wo, bo):
  B = GG.shape[0]
  D = 64
  H3 = W3.shape[1]
  bb = 2048
  grid = B // bb

  def body(gg_r, xx_r, W1_r, b1_r, W2_r, b2_r, W3_r, b3_r, Wo_r, bo_r, o_r):
    dot = functools.partial(jnp.dot, precision=lax.Precision.HIGHEST,
                            preferred_element_type=jnp.float32)
    h = jnp.maximum(dot(xx_r[...], W1_r[...]) + b1_r[...], 0.0)
    h = jnp.maximum(dot(h, W2_r[...]) + b2_r[...], 0.0)
    h = jnp.maximum(dot(h, W3_r[...]) + b3_r[...], 0.0)
    G2 = gg_r[...]
    g = G2[:, 0:D] * G2[:, D:2 * D]
    logit = dot(g, Wo_r[0:D, :]) + dot(h, Wo_r[D:D + H3, :]) + bo_r[...]
    o_r[...] = jax.nn.sigmoid(logit[:, 0])

  row = pl.BlockSpec((bb, 128), lambda i: (i, 0))
  full = lambda a: pl.BlockSpec(a.shape, lambda i: (0,) * a.ndim)
  return pl.pallas_call(
      body,
      grid=(grid,),
      in_specs=[row, row,
                full(W1), full(b1), full(W2), full(b2),
                full(W3), full(b3), full(Wo), full(bo)],
      out_specs=pl.BlockSpec((bb,), lambda i: (i,)),
      out_shape=jax.ShapeDtypeStruct((B,), jnp.float32),
  )(GG, XX, W1, b1, W2, b2, W3, b3, Wo, bo)


def kernel(user_indices, item_indices, user_emb_gmf, item_emb_gmf,
           user_emb_mlp, item_emb_mlp, W1, b1, W2, b2, W3, b3, Wo, bo):
  B = user_indices.shape[0]
  remap = lambda i: jnp.where(i < _TH, 2 * i, 2 * (i - _TH) + 1)
  cu = remap(user_indices)
  ci = remap(item_indices) + 2 * _TH
  cidx = jnp.stack((cu, ci), axis=-1).reshape(2 * B)

  pg = _tc_pack_pair(user_emb_gmf.T, item_emb_gmf.T)
  GG = _sc_gather_pair(cidx, pg.reshape(4 * _TH, 64))
  pm = _tc_pack_pair(user_emb_mlp.T, item_emb_mlp.T)
  XX = _sc_gather_pair(cidx, pm.reshape(4 * _TH, 64))

  return _tc_mlp(GG.reshape(B, 128), XX.reshape(B, 128),
                 W1, b1, W2, b2, W3, b3, Wo, bo)


# speedup vs baseline: 1.2151x; 1.2151x over previous
import functools

import jax
import jax.numpy as jnp
from jax import lax
from jax.experimental import pallas as pl
from jax.experimental.pallas import tpu as pltpu
from jax.experimental.pallas import tpu_sc as plsc

_CHUNK = 128
_TW = 2048
_TH = 25 * _TW


def _tc_pack4(tables):
  G = _TH // _TW
  last_blk = (tables[0].shape[1] - 1) // _TW

  def body(*refs):
    ins, out_r = refs[:8], refs[8]
    for t in range(4):
      lo, hi = ins[2 * t][...], ins[2 * t + 1][...]
      out_r[t, :, :] = jnp.concatenate([lo.T, hi.T], axis=1)

  lo_spec = pl.BlockSpec((64, _TW), lambda i: (0, i))
  hi_spec = pl.BlockSpec((64, _TW), lambda i: (0, jnp.minimum(i + G, last_blk)))
  return pl.pallas_call(
      body,
      grid=(G,),
      in_specs=[s for _ in range(4) for s in (lo_spec, hi_spec)],
      out_specs=pl.BlockSpec((4, _TW, 128), lambda i: (0, i, 0)),
      out_shape=jax.ShapeDtypeStruct((4, _TH, 128), jnp.float32),
  )(*[t for tbl in tables for t in (tbl, tbl)])


def _sc_gather(cidx, table):
  B4 = cidx.shape[0]
  D = table.shape[1]
  info = plsc.get_sparse_core_info()
  NC, NS = info.num_cores, info.num_subcores
  NW = NC * NS
  assert B4 % (NW * _CHUNK) == 0
  bpw = B4 // NW
  ch = bpw // _CHUNK
  GRP = 4
  rows_g = GRP * _CHUNK
  ngrp = ch // GRP

  cidx2 = cidx.reshape(NW * ch, _CHUNK)

  mesh = plsc.VectorSubcoreMesh(core_axis_name="c", subcore_axis_name="s")

  @functools.partial(
      pl.kernel,
      out_type=jax.ShapeDtypeStruct((B4, D), jnp.float32),
      mesh=mesh,
      compiler_params=pltpu.CompilerParams(use_tc_tiling_on_sc=False),
      scratch_types=[
          pltpu.VMEM((ch, _CHUNK), jnp.int32),
          pltpu.VMEM((rows_g, D), jnp.float32),
          pltpu.VMEM((rows_g, D), jnp.float32),
          pltpu.SemaphoreType.DMA,
          pltpu.SemaphoreType.DMA,
      ],
  )
  def k(cix, tbl, out, idx_v, rows0, rows1, sem0, sem1):
    wid = lax.axis_index("s") * NC + lax.axis_index("c")
    base = wid * bpw
    pltpu.sync_copy(cix.at[pl.ds(wid * ch, ch)], idx_v)

    bufs = (rows0, rows1)
    sems = (sem0, sem1)

    def fire(g):
      buf, sem = bufs[g % 2], sems[g % 2]
      return [
          pltpu.async_copy(tbl.at[idx_v.at[g * GRP + j]],
                           buf.at[pl.ds(j * _CHUNK, _CHUNK)], sem)
          for j in range(GRP)
      ]

    descs = [None] * ngrp
    descs[0] = fire(0)
    if ngrp > 1:
      descs[1] = fire(1)
    for g in range(ngrp):
      for d in descs[g]:
        d.wait()
      pltpu.sync_copy(bufs[g % 2], out.at[pl.ds(base + g * rows_g, rows_g)])
      if g + 2 < ngrp:
        descs[g + 2] = fire(g + 2)

  return k(cidx2, table)


def _tc_mlp(R, W1, b1, W2, b2, W3, b3, Wo, bo):
  B = R.shape[0]
  D = 64
  H3 = W3.shape[1]
  bb = 2048
  grid = B // bb

  def body(r_r, W1_r, b1_r, W2_r, b2_r, W3_r, b3_r, Wo_r, bo_r, o_r):
    dot = functools.partial(jnp.dot, preferred_element_type=jnp.float32)
    R2 = r_r[...]
    h = jnp.maximum(dot(R2[:, 128:256], W1_r[...]) + b1_r[...], 0.0)
    h = jnp.maximum(dot(h, W2_r[...]) + b2_r[...], 0.0)
    h = jnp.maximum(dot(h, W3_r[...]) + b3_r[...], 0.0)
    g = R2[:, 0:D] * R2[:, D:2 * D]
    logit = dot(g, Wo_r[0:D, :]) + dot(h, Wo_r[D:D + H3, :]) + bo_r[...]
    o_r[...] = jax.nn.sigmoid(logit[:, 0])

  row = pl.BlockSpec((bb, 256), lambda i: (i, 0))
  full = lambda a: pl.BlockSpec(a.shape, lambda i: (0,) * a.ndim)
  return pl.pallas_call(
      body,
      grid=(grid,),
      in_specs=[row,
                full(W1), full(b1), full(W2), full(b2),
                full(W3), full(b3), full(Wo), full(bo)],
      out_specs=pl.BlockSpec((bb,), lambda i: (i,)),
      out_shape=jax.ShapeDtypeStruct((B,), jnp.float32),
  )(R, W1, b1, W2, b2, W3, b3, Wo, bo)


def kernel(user_indices, item_indices, user_emb_gmf, item_emb_gmf,
           user_emb_mlp, item_emb_mlp, W1, b1, W2, b2, W3, b3, Wo, bo):
  B = user_indices.shape[0]
  remap = lambda i: jnp.where(i < _TH, 2 * i, 2 * (i - _TH) + 1)
  cu, ci = remap(user_indices), remap(item_indices)
  H2 = 2 * _TH
  cidx = jnp.stack((cu, ci + H2, cu + 2 * H2, ci + 3 * H2),
                   axis=-1).reshape(4 * B)

  packed = _tc_pack4(
      (user_emb_gmf.T, item_emb_gmf.T, user_emb_mlp.T, item_emb_mlp.T))
  R = _sc_gather(cidx, packed.reshape(8 * _TH, 64))
  return _tc_mlp(R.reshape(B, 256), W1, b1, W2, b2, W3, b3, Wo, bo)
